# Initial kernel scaffold; baseline (speedup 1.0000x reference)
import jax
import jax.numpy as jnp
from jax.experimental import pallas as pl


def kernel(inputs, labels, W, b):
    raise NotImplementedError("write your pallas kernel here")


# speedup vs baseline: 12.2105x; 12.2105x over previous
import functools

import jax
import jax.numpy as jnp
from jax.experimental import pallas as pl
from jax.experimental.pallas import tpu as pltpu

N, D_IN, D_OUT, C = 8192, 512, 512, 64
CHUNK = 512
K = N // CHUNK
BIG = 2147483647


def _tc_body(labels_ref, x_ref, W_ref, b_ref, out_ref, sums_ref, cnt_ref, fi_ref):
    k = pl.program_id(0)

    @pl.when(k == 0)
    def _init():
        sums_ref[...] = jnp.zeros_like(sums_ref)
        cnt_ref[...] = jnp.zeros_like(cnt_ref)
        fi_ref[...] = jnp.full_like(fi_ref, BIG)

    labels = labels_ref[0]
    lab_b = jnp.broadcast_to(labels, (C, CHUNK))
    class_ids = jax.lax.broadcasted_iota(jnp.int32, (C, CHUNK), 0)
    onehot = (lab_b == class_ids)

    x = x_ref[...]
    sums_ref[...] += jax.lax.dot_general(
        onehot.astype(jnp.float32), x,
        dimension_numbers=(((1,), (0,)), ((), ())),
        preferred_element_type=jnp.float32, precision=jax.lax.Precision.HIGHEST,
    )

    cnt = jnp.sum(onehot.astype(jnp.float32), axis=1, keepdims=True)
    cnt_ref[...] += jnp.broadcast_to(cnt, (C, 128))

    row_idx = k * CHUNK + jax.lax.broadcasted_iota(jnp.int32, (C, CHUNK), 1)
    masked = jnp.where(onehot, row_idx, BIG)
    chunk_min = jnp.min(masked, axis=1, keepdims=True)
    fi_ref[...] = jnp.minimum(fi_ref[...], jnp.broadcast_to(chunk_min, (C, 128)))

    @pl.when(k == K - 1)
    def _finish():
        cnt_col = cnt_ref[:, :1]
        means = sums_ref[...] / jnp.broadcast_to(cnt_col, (C, D_IN))
        proto = jax.lax.dot_general(
            means, W_ref[...],
            dimension_numbers=(((1,), (0,)), ((), ())),
            preferred_element_type=jnp.float32, precision=jax.lax.Precision.HIGHEST,
        ) + b_ref[...]

        fi_col = fi_ref[:, :1].astype(jnp.float32)
        eye = (jax.lax.broadcasted_iota(jnp.int32, (C, C), 0)
               == jax.lax.broadcasted_iota(jnp.int32, (C, C), 1)).astype(jnp.float32)
        fi_rowv = jax.lax.dot_general(
            fi_col, eye, dimension_numbers=(((0,), (0,)), ((), ())),
            preferred_element_type=jnp.float32, precision=jax.lax.Precision.HIGHEST,
        )
        fi_lanes = jnp.broadcast_to(fi_rowv, (C, C))
        fi_subl = jnp.broadcast_to(fi_col, (C, C))
        lane_id = jax.lax.broadcasted_iota(jnp.int32, (C, C), 1)
        subl_id = jax.lax.broadcasted_iota(jnp.int32, (C, C), 0)
        less = (fi_lanes < fi_subl) | ((fi_lanes == fi_subl) & (lane_id < subl_id))
        rank_col = jnp.sum(less.astype(jnp.float32), axis=1, keepdims=True)
        rank_rowv = jax.lax.dot_general(
            rank_col, eye, dimension_numbers=(((0,), (0,)), ((), ())),
            preferred_element_type=jnp.float32, precision=jax.lax.Precision.HIGHEST,
        )
        perm = (jnp.broadcast_to(rank_rowv, (C, C))
                == subl_id.astype(jnp.float32)).astype(jnp.float32)
        out_ref[...] = jax.lax.dot_general(
            perm, proto, dimension_numbers=(((1,), (0,)), ((), ())),
            preferred_element_type=jnp.float32, precision=jax.lax.Precision.HIGHEST,
        )


@functools.partial(jax.jit, static_argnames=())
def kernel(inputs, labels, W, b):
    labels3d = labels.reshape(K, 1, CHUNK)
    b2d = b.reshape(1, D_OUT)
    out = pl.pallas_call(
        _tc_body,
        grid=(K,),
        in_specs=[
            pl.BlockSpec((1, 1, CHUNK), lambda k: (k, 0, 0)),
            pl.BlockSpec((CHUNK, D_IN), lambda k: (k, 0)),
            pl.BlockSpec((D_IN, D_OUT), lambda k: (0, 0)),
            pl.BlockSpec((1, D_OUT), lambda k: (0, 0)),
        ],
        out_specs=pl.BlockSpec((C, D_OUT), lambda k: (0, 0)),
        out_shape=jax.ShapeDtypeStruct((C, D_OUT), jnp.float32),
        scratch_shapes=[
            pltpu.VMEM((C, D_IN), jnp.float32),
            pltpu.VMEM((C, 128), jnp.float32),
            pltpu.VMEM((C, 128), jnp.int32),
        ],
        compiler_params=pltpu.CompilerParams(
            dimension_semantics=("arbitrary",),
        ),
    )(labels3d, inputs, W, b2d)
    return out


# speedup vs baseline: 14.5157x; 1.1888x over previous
import functools

import jax
import jax.numpy as jnp
from jax.experimental import pallas as pl
from jax.experimental.pallas import tpu as pltpu

N, D_IN, D_OUT, C = 8192, 512, 512, 64
CHUNK = 512
K = N // CHUNK
BIG = 2147483647


def _tc_body(labels_ref, x_ref, W_ref, b_ref, out_ref, sums_ref, cnt_ref, fi_ref):
    k = pl.program_id(0)

    @pl.when(k == 0)
    def _init():
        sums_ref[...] = jnp.zeros_like(sums_ref)
        cnt_ref[...] = jnp.zeros_like(cnt_ref)
        fi_ref[...] = jnp.full_like(fi_ref, BIG)

    labels = labels_ref[0]
    lab_b = jnp.broadcast_to(labels, (C, CHUNK))
    class_ids = jax.lax.broadcasted_iota(jnp.int32, (C, CHUNK), 0)
    onehot = (lab_b == class_ids)

    x = x_ref[...]
    x_hi = x.astype(jnp.bfloat16)
    x_lo = (x - x_hi.astype(jnp.float32)).astype(jnp.bfloat16)
    oh_bf = onehot.astype(jnp.bfloat16)
    dn = (((1,), (0,)), ((), ()))
    sums_ref[...] += (
        jax.lax.dot_general(oh_bf, x_hi, dimension_numbers=dn,
                            preferred_element_type=jnp.float32)
        + jax.lax.dot_general(oh_bf, x_lo, dimension_numbers=dn,
                              preferred_element_type=jnp.float32)
    )

    cnt = jnp.sum(onehot.astype(jnp.float32), axis=1, keepdims=True)
    cnt_ref[...] += jnp.broadcast_to(cnt, (C, 128))

    row_idx = k * CHUNK + jax.lax.broadcasted_iota(jnp.int32, (C, CHUNK), 1)
    masked = jnp.where(onehot, row_idx, BIG)
    chunk_min = jnp.min(masked, axis=1, keepdims=True)
    fi_ref[...] = jnp.minimum(fi_ref[...], jnp.broadcast_to(chunk_min, (C, 128)))

    @pl.when(k == K - 1)
    def _finish():
        cnt_col = cnt_ref[:, :1]
        means = sums_ref[...] / jnp.broadcast_to(cnt_col, (C, D_IN))
        proto = jax.lax.dot_general(
            means, W_ref[...],
            dimension_numbers=(((1,), (0,)), ((), ())),
            preferred_element_type=jnp.float32, precision=jax.lax.Precision.HIGHEST,
        ) + b_ref[...]

        fi_col = fi_ref[:, :1].astype(jnp.float32)
        eye = (jax.lax.broadcasted_iota(jnp.int32, (C, C), 0)
               == jax.lax.broadcasted_iota(jnp.int32, (C, C), 1)).astype(jnp.float32)
        fi_rowv = jax.lax.dot_general(
            fi_col, eye, dimension_numbers=(((0,), (0,)), ((), ())),
            preferred_element_type=jnp.float32, precision=jax.lax.Precision.HIGHEST,
        )
        fi_lanes = jnp.broadcast_to(fi_rowv, (C, C))
        fi_subl = jnp.broadcast_to(fi_col, (C, C))
        lane_id = jax.lax.broadcasted_iota(jnp.int32, (C, C), 1)
        subl_id = jax.lax.broadcasted_iota(jnp.int32, (C, C), 0)
        less = (fi_lanes < fi_subl) | ((fi_lanes == fi_subl) & (lane_id < subl_id))
        rank_col = jnp.sum(less.astype(jnp.float32), axis=1, keepdims=True)
        rank_rowv = jax.lax.dot_general(
            rank_col, eye, dimension_numbers=(((0,), (0,)), ((), ())),
            preferred_element_type=jnp.float32, precision=jax.lax.Precision.HIGHEST,
        )
        perm = (jnp.broadcast_to(rank_rowv, (C, C))
                == subl_id.astype(jnp.float32)).astype(jnp.float32)
        out_ref[...] = jax.lax.dot_general(
            perm, proto, dimension_numbers=(((1,), (0,)), ((), ())),
            preferred_element_type=jnp.float32, precision=jax.lax.Precision.HIGHEST,
        )


@functools.partial(jax.jit, static_argnames=())
def kernel(inputs, labels, W, b):
    labels3d = labels.reshape(K, 1, CHUNK)
    b2d = b.reshape(1, D_OUT)
    out = pl.pallas_call(
        _tc_body,
        grid=(K,),
        in_specs=[
            pl.BlockSpec((1, 1, CHUNK), lambda k: (k, 0, 0)),
            pl.BlockSpec((CHUNK, D_IN), lambda k: (k, 0)),
            pl.BlockSpec((D_IN, D_OUT), lambda k: (0, 0)),
            pl.BlockSpec((1, D_OUT), lambda k: (0, 0)),
        ],
        out_specs=pl.BlockSpec((C, D_OUT), lambda k: (0, 0)),
        out_shape=jax.ShapeDtypeStruct((C, D_OUT), jnp.float32),
        scratch_shapes=[
            pltpu.VMEM((C, D_IN), jnp.float32),
            pltpu.VMEM((C, 128), jnp.float32),
            pltpu.VMEM((C, 128), jnp.int32),
        ],
        compiler_params=pltpu.CompilerParams(
            dimension_semantics=("arbitrary",),
        ),
    )(labels3d, inputs, W, b2d)
    return out
